# Initial kernel scaffold; baseline (speedup 1.0000x reference)
import jax
import jax.numpy as jnp
from jax.experimental import pallas as pl


def kernel(x, edge_index, edge_label_index, W_l, b_l, W_r, b_r, att, bias, gamma, beta):
    raise NotImplementedError("write your pallas kernel here")


# speedup vs baseline: 3.7706x; 3.7706x over previous
import dataclasses
import functools

import jax
import jax.numpy as jnp
from jax import lax
from jax.experimental import pallas as pl
from jax.experimental.pallas import tpu as pltpu
from jax.experimental.pallas import tpu_sc as plsc

N = 10000
D = 128
H = 256
E = 320000
EL = 320000
ET = E + N
G = 128
NTILE = 16
ET_PAD = 16 * G * 162
EL_PAD = 16 * G * 157
NPAD = 10240
NEG = -3.0e38

_mesh = plsc.VectorSubcoreMesh(core_axis_name="c", subcore_axis_name="s")

_cp = pltpu.CompilerParams()
if "needs_layout_passes" in pltpu.CompilerParams.__dataclass_fields__:
    _cp = dataclasses.replace(_cp, needs_layout_passes=False)


def _dense_body(x_ref, wl_ref, bl_ref, wr_ref, br_ref, ol_ref, or_ref):
    xb = x_ref[...]
    a = jnp.dot(xb, wl_ref[...], preferred_element_type=jnp.float32) + bl_ref[...]
    ol_ref[0] = a[:, :128]
    ol_ref[1] = a[:, 128:]
    b = jnp.dot(xb, wr_ref[...], preferred_element_type=jnp.float32) + br_ref[...]
    or_ref[0] = b[:, :128]
    or_ref[1] = b[:, 128:]


def _dense_pre(x, W_l, b_l, W_r, b_r):
    Bn = 1000
    return pl.pallas_call(
        _dense_body,
        grid=(N // Bn,),
        in_specs=[
            pl.BlockSpec((Bn, D), lambda i: (i, 0)),
            pl.BlockSpec((D, H), lambda i: (0, 0)),
            pl.BlockSpec((1, H), lambda i: (0, 0)),
            pl.BlockSpec((D, H), lambda i: (0, 0)),
            pl.BlockSpec((1, H), lambda i: (0, 0)),
        ],
        out_specs=[
            pl.BlockSpec((2, Bn, 128), lambda i: (0, i, 0)),
            pl.BlockSpec((2, Bn, 128), lambda i: (0, i, 0)),
        ],
        out_shape=[
            jax.ShapeDtypeStruct((2, N, 128), jnp.float32),
            jax.ShapeDtypeStruct((2, N, 128), jnp.float32),
        ],
    )(x, W_l, b_l.reshape(1, H), W_r, b_r.reshape(1, H))


def _sc_logits(xl2, xr2, src, dst, att):
    groups = ET_PAD // (NTILE * G)

    @functools.partial(
        pl.kernel,
        mesh=_mesh,
        compiler_params=_cp,
        out_type=jax.ShapeDtypeStruct((2 * ET_PAD,), jnp.float32),
        scratch_types=[
            pltpu.VMEM((G,), jnp.int32),
            pltpu.VMEM((G,), jnp.int32),
            pltpu.VMEM((G, 128), jnp.float32),
            pltpu.VMEM((G, 128), jnp.float32),
            pltpu.VMEM((128,), jnp.float32),
            pltpu.VMEM((G * 16,), jnp.float32),
            pltpu.VMEM((G,), jnp.float32),
            pltpu.SemaphoreType.DMA,
            pltpu.SemaphoreType.DMA,
        ],
    )
    def k(xl_h, xr_h, src_h, dst_h, att_h, plog_h,
          sadj, dadj, urows, vrows, attv, accb, plogv, sem1, sem2):
        c = lax.axis_index("c")
        t = lax.axis_index("s")
        cN = c * N
        pltpu.sync_copy(att_h.at[pl.ds(c * 128, 128)], attv)
        tile_base = t * (groups * G)

        @pl.loop(0, groups)
        def _(g):
            base = tile_base + g * G
            pltpu.sync_copy(src_h.at[pl.ds(base, G)], sadj)
            pltpu.sync_copy(dst_h.at[pl.ds(base, G)], dadj)
            for j in range(G // 16):
                sl = pl.ds(j * 16, 16)
                sadj[sl] = sadj[sl] + cN
                dadj[sl] = dadj[sl] + cN
            cp1 = pltpu.async_copy(xl_h.at[sadj], urows, sem1)
            cp2 = pltpu.async_copy(xr_h.at[dadj], vrows, sem2)
            cp1.wait()
            cp2.wait()

            @pl.loop(0, G)
            def _(e):
                acc = jnp.zeros((16,), jnp.float32)
                for hc in range(8):
                    sl = pl.ds(hc * 16, 16)
                    tv = urows[e, sl] + vrows[e, sl]
                    lr = jnp.maximum(tv, 0.2 * tv)
                    acc = acc + lr * attv[sl]
                accb[pl.ds(e * 16, 16)] = acc

            @pl.loop(0, G // 16)
            def _(eg):
                lanes = lax.iota(jnp.int32, 16) * 16 + eg * 256
                lg = jnp.zeros((16,), jnp.float32)
                for j in range(16):
                    lg = lg + plsc.load_gather(accb, [lanes + j])
                plogv[pl.ds(eg * 16, 16)] = lg

            pltpu.sync_copy(plogv, plog_h.at[pl.ds(c * ET_PAD + base, G)])

    return k(xl2, xr2, src, dst, att)


def _combine_max_body(p_ref, lo_ref, m_ref):
    s = p_ref[0] + p_ref[1]
    r = lax.broadcasted_iota(jnp.int32, s.shape, 0)
    cidx = lax.broadcasted_iota(jnp.int32, s.shape, 1)
    gidx = r * 128 + cidx
    s = jnp.where(gidx < ET, s, NEG)
    lo_ref[...] = s
    m_ref[...] = jnp.broadcast_to(jnp.max(s), (8, 128))


def _combine_max(plog3):
    return pl.pallas_call(
        _combine_max_body,
        out_shape=[
            jax.ShapeDtypeStruct((ET_PAD // 128, 128), jnp.float32),
            jax.ShapeDtypeStruct((8, 128), jnp.float32),
        ],
    )(plog3)


def _sc_aggregate(xl2, logits, src, dst, m16):
    groups = ET_PAD // (NTILE * G)
    rows_per_tile = NPAD // NTILE

    @functools.partial(
        pl.kernel,
        mesh=_mesh,
        compiler_params=_cp,
        out_type=(
            jax.ShapeDtypeStruct((2 * NPAD, 128), jnp.float32),
            jax.ShapeDtypeStruct((NPAD,), jnp.float32),
        ),
        scratch_types=[
            pltpu.VMEM_SHARED((NPAD, 128), jnp.float32),
            pltpu.VMEM_SHARED((NPAD,), jnp.float32),
            pltpu.VMEM((G,), jnp.int32),
            pltpu.VMEM((G,), jnp.int32),
            pltpu.VMEM((G,), jnp.float32),
            pltpu.VMEM((G,), jnp.float32),
            pltpu.VMEM((G, 128), jnp.float32),
            pltpu.VMEM((128, 128), jnp.float32),
            pltpu.VMEM((rows_per_tile,), jnp.float32),
            pltpu.VMEM((16,), jnp.float32),
            pltpu.SemaphoreType.DMA,
        ],
    )
    def k(xl_h, lg_h, src_h, dst_h, m_h, out_h, den_h,
          oacc, dacc, sadj, didx, lgv, exv, rows, zb, zd, mv, sem):
        c = lax.axis_index("c")
        t = lax.axis_index("s")
        cN = c * N
        pltpu.sync_copy(m_h, mv)
        mvec = mv[...]

        @pl.loop(0, 128)
        def _(r):
            for j in range(8):
                zb[r, pl.ds(j * 16, 16)] = jnp.zeros((16,), jnp.float32)

        @pl.loop(0, rows_per_tile // 16)
        def _(r):
            zd[pl.ds(r * 16, 16)] = jnp.zeros((16,), jnp.float32)

        r0 = t * rows_per_tile
        for kk in range(rows_per_tile // 128):
            pltpu.sync_copy(zb, oacc.at[pl.ds(r0 + kk * 128, 128)])
        pltpu.sync_copy(zd, dacc.at[pl.ds(r0, rows_per_tile)])
        plsc.subcore_barrier()

        tile_base = t * (groups * G)

        @pl.loop(0, groups)
        def _(g):
            base = tile_base + g * G
            pltpu.sync_copy(src_h.at[pl.ds(base, G)], sadj)
            pltpu.sync_copy(dst_h.at[pl.ds(base, G)], didx)
            pltpu.sync_copy(lg_h.at[pl.ds(base, G)], lgv)
            for j in range(G // 16):
                sl = pl.ds(j * 16, 16)
                sadj[sl] = sadj[sl] + cN
                exv[sl] = jnp.exp(lgv[sl] - mvec)
            pltpu.async_copy(xl_h.at[sadj], rows, sem).wait()

            @pl.loop(0, G)
            def _(e):
                ee = plsc.load_gather(exv, [jnp.full((16,), e, jnp.int32)])
                for hc in range(8):
                    sl = pl.ds(hc * 16, 16)
                    rows[e, sl] = rows[e, sl] * ee

            pltpu.sync_copy(rows, oacc.at[didx], add=True)
            pltpu.sync_copy(exv, dacc.at[didx], add=True)

        plsc.subcore_barrier()
        pltpu.sync_copy(oacc.at[pl.ds(r0, rows_per_tile)],
                        out_h.at[pl.ds(c * NPAD + r0, rows_per_tile)])

        @pl.when(c == 0)
        def _():
            pltpu.sync_copy(dacc.at[pl.ds(r0, rows_per_tile)],
                            den_h.at[pl.ds(r0, rows_per_tile)])

    return k(xl2, logits, src, dst, m16)


def _bn_stats_body(o_ref, d_ref, b_ref, e_ref, s_ref, ss_ref):
    i = pl.program_id(1)
    o = o_ref[0] / (d_ref[0, 0][:, None] + 1e-16) + b_ref[0, 0][None, :]
    r = lax.broadcasted_iota(jnp.int32, o.shape, 0) + i * o.shape[0]
    o = jnp.where(r < N, o, 0.0)
    e_ref[0] = o

    @pl.when(i == 0)
    def _():
        s_ref[0, 0] = jnp.zeros((128,), jnp.float32)
        ss_ref[0, 0] = jnp.zeros((128,), jnp.float32)
    s_ref[0, 0] += jnp.sum(o, axis=0)
    ss_ref[0, 0] += jnp.sum(o * o, axis=0)


def _bn_stats(out3, denom2, bias2):
    Bn = 1024
    return pl.pallas_call(
        _bn_stats_body,
        grid=(2, NPAD // Bn),
        in_specs=[
            pl.BlockSpec((1, Bn, 128), lambda h, i: (h, i, 0)),
            pl.BlockSpec((1, 1, Bn), lambda h, i: (i, 0, 0)),
            pl.BlockSpec((1, 1, 128), lambda h, i: (h, 0, 0)),
        ],
        out_specs=[
            pl.BlockSpec((1, Bn, 128), lambda h, i: (h, i, 0)),
            pl.BlockSpec((1, 1, 128), lambda h, i: (h, 0, 0)),
            pl.BlockSpec((1, 1, 128), lambda h, i: (h, 0, 0)),
        ],
        out_shape=[
            jax.ShapeDtypeStruct((2, NPAD, 128), jnp.float32),
            jax.ShapeDtypeStruct((2, 1, 128), jnp.float32),
            jax.ShapeDtypeStruct((2, 1, 128), jnp.float32),
        ],
    )(out3, denom2, bias2.reshape(2, 1, 128))


def _bn_apply_body(e_ref, s_ref, ss_ref, g_ref, b_ref, o_ref):
    mean = s_ref[0, 0] / N
    var = ss_ref[0, 0] / N - mean * mean
    scale = g_ref[0, 0][None, :] * lax.rsqrt(var + 1e-5)[None, :]
    o = (e_ref[0] - mean[None, :]) * scale + b_ref[0, 0][None, :]
    o_ref[0] = jnp.maximum(o, 0.0)


def _bn_apply(emb3, s, ss, gamma2, beta2):
    Bn = 1024
    return pl.pallas_call(
        _bn_apply_body,
        grid=(2, NPAD // Bn),
        in_specs=[
            pl.BlockSpec((1, Bn, 128), lambda h, i: (h, i, 0)),
            pl.BlockSpec((1, 1, 128), lambda h, i: (h, 0, 0)),
            pl.BlockSpec((1, 1, 128), lambda h, i: (h, 0, 0)),
            pl.BlockSpec((1, 1, 128), lambda h, i: (h, 0, 0)),
            pl.BlockSpec((1, 1, 128), lambda h, i: (h, 0, 0)),
        ],
        out_specs=pl.BlockSpec((1, Bn, 128), lambda h, i: (h, i, 0)),
        out_shape=jax.ShapeDtypeStruct((2, NPAD, 128), jnp.float32),
    )(emb3, s, ss, gamma2.reshape(2, 1, 128), beta2.reshape(2, 1, 128))


def _sc_scores(embs2, sl_idx, tl_idx):
    groups = EL_PAD // (NTILE * G)

    @functools.partial(
        pl.kernel,
        mesh=_mesh,
        compiler_params=_cp,
        out_type=jax.ShapeDtypeStruct((2 * EL_PAD,), jnp.float32),
        scratch_types=[
            pltpu.VMEM((G,), jnp.int32),
            pltpu.VMEM((G,), jnp.int32),
            pltpu.VMEM((G, 128), jnp.float32),
            pltpu.VMEM((G, 128), jnp.float32),
            pltpu.VMEM((G * 16,), jnp.float32),
            pltpu.VMEM((G,), jnp.float32),
            pltpu.SemaphoreType.DMA,
            pltpu.SemaphoreType.DMA,
        ],
    )
    def k(emb_h, s_h, t_h, ps_h,
          sadj, tadj, srows, trows, accb, psv, sem1, sem2):
        c = lax.axis_index("c")
        t = lax.axis_index("s")
        cN = c * NPAD
        tile_base = t * (groups * G)

        @pl.loop(0, groups)
        def _(g):
            base = tile_base + g * G
            pltpu.sync_copy(s_h.at[pl.ds(base, G)], sadj)
            pltpu.sync_copy(t_h.at[pl.ds(base, G)], tadj)
            for j in range(G // 16):
                sl = pl.ds(j * 16, 16)
                sadj[sl] = sadj[sl] + cN
                tadj[sl] = tadj[sl] + cN
            cp1 = pltpu.async_copy(emb_h.at[sadj], srows, sem1)
            cp2 = pltpu.async_copy(emb_h.at[tadj], trows, sem2)
            cp1.wait()
            cp2.wait()

            @pl.loop(0, G)
            def _(e):
                acc = jnp.zeros((16,), jnp.float32)
                for hc in range(8):
                    sl = pl.ds(hc * 16, 16)
                    acc = acc + srows[e, sl] * trows[e, sl]
                accb[pl.ds(e * 16, 16)] = acc

            @pl.loop(0, G // 16)
            def _(eg):
                lanes = lax.iota(jnp.int32, 16) * 16 + eg * 256
                ps = jnp.zeros((16,), jnp.float32)
                for j in range(16):
                    ps = ps + plsc.load_gather(accb, [lanes + j])
                psv[pl.ds(eg * 16, 16)] = ps

            pltpu.sync_copy(psv, ps_h.at[pl.ds(c * EL_PAD + base, G)])

    return k(embs2, sl_idx, tl_idx)


def _combine_scores_body(p_ref, o_ref):
    o_ref[...] = p_ref[0] + p_ref[1]


def _combine_scores(ps3):
    return pl.pallas_call(
        _combine_scores_body,
        out_shape=jax.ShapeDtypeStruct((EL_PAD // 128, 128), jnp.float32),
    )(ps3)


@jax.jit
def _run(x, edge_index, edge_label_index, W_l, b_l, W_r, b_r, att, bias,
         gamma, beta):
    loop = jnp.arange(N, dtype=jnp.int32)
    pad_e = jnp.zeros((ET_PAD - ET,), jnp.int32)
    src_f = jnp.concatenate([edge_index[0], loop, pad_e])
    dst_f = jnp.concatenate([edge_index[1], loop, pad_e])

    xl3, xr3 = _dense_pre(x, W_l, b_l, W_r, b_r)
    xl2 = xl3.reshape(2 * N, 128)
    xr2 = xr3.reshape(2 * N, 128)

    plog = _sc_logits(xl2, xr2, src_f, dst_f, att)
    logits3, m = _combine_max(plog.reshape(2, ET_PAD // 128, 128))
    m16 = jnp.full((16,), m[0, 0], jnp.float32)

    out2, denom = _sc_aggregate(xl2, logits3.reshape(ET_PAD), src_f, dst_f, m16)
    emb3, s, ss = _bn_stats(out2.reshape(2, NPAD, 128),
                            denom.reshape(NPAD // 1024, 1, 1024),
                            bias.reshape(2, 128))
    embs3 = _bn_apply(emb3, s, ss, gamma.reshape(2, 128), beta.reshape(2, 128))

    pad_l = jnp.zeros((EL_PAD - EL,), jnp.int32)
    sl_idx = jnp.concatenate([edge_label_index[0], pad_l])
    tl_idx = jnp.concatenate([edge_label_index[1], pad_l])
    ps = _sc_scores(embs3.reshape(2 * NPAD, 128), sl_idx, tl_idx)
    sc3 = _combine_scores(ps.reshape(2, EL_PAD // 128, 128))
    return sc3.reshape(EL_PAD)[:EL]


def kernel(x, edge_index, edge_label_index, W_l, b_l, W_r, b_r, att, bias,
           gamma, beta):
    return _run(x, edge_index, edge_label_index, W_l, b_l, W_r, b_r, att,
                bias, gamma, beta)


# speedup vs baseline: 4.5212x; 1.1991x over previous
import dataclasses
import functools

import jax
import jax.numpy as jnp
from jax import lax
from jax.experimental import pallas as pl
from jax.experimental.pallas import tpu as pltpu
from jax.experimental.pallas import tpu_sc as plsc

N = 10000
D = 128
H = 256
E = 320000
EL = 320000
ET = E + N
G = 128
NTILE = 16
ET_PAD = 16 * G * 162
EL_PAD = 16 * G * 157
NPAD = 10240
NEG = -3.0e38

_mesh = plsc.VectorSubcoreMesh(core_axis_name="c", subcore_axis_name="s")

_cp = pltpu.CompilerParams()
if "needs_layout_passes" in pltpu.CompilerParams.__dataclass_fields__:
    _cp = dataclasses.replace(_cp, needs_layout_passes=False)


def _dense_body(x_ref, wl_ref, bl_ref, wr_ref, br_ref, ol_ref, or_ref):
    xb = x_ref[...]
    a = jnp.dot(xb, wl_ref[...], preferred_element_type=jnp.float32) + bl_ref[...]
    ol_ref[0] = a[:, :128]
    ol_ref[1] = a[:, 128:]
    b = jnp.dot(xb, wr_ref[...], preferred_element_type=jnp.float32) + br_ref[...]
    or_ref[0] = b[:, :128]
    or_ref[1] = b[:, 128:]


def _dense_pre(x, W_l, b_l, W_r, b_r):
    Bn = 1000
    return pl.pallas_call(
        _dense_body,
        grid=(N // Bn,),
        in_specs=[
            pl.BlockSpec((Bn, D), lambda i: (i, 0)),
            pl.BlockSpec((D, H), lambda i: (0, 0)),
            pl.BlockSpec((1, H), lambda i: (0, 0)),
            pl.BlockSpec((D, H), lambda i: (0, 0)),
            pl.BlockSpec((1, H), lambda i: (0, 0)),
        ],
        out_specs=[
            pl.BlockSpec((2, Bn, 128), lambda i: (0, i, 0)),
            pl.BlockSpec((2, Bn, 128), lambda i: (0, i, 0)),
        ],
        out_shape=[
            jax.ShapeDtypeStruct((2, N, 128), jnp.float32),
            jax.ShapeDtypeStruct((2, N, 128), jnp.float32),
        ],
    )(x, W_l, b_l.reshape(1, H), W_r, b_r.reshape(1, H))


def _sc_logits(xl2, xr2, src, dst, att):
    groups = ET_PAD // (NTILE * G)
    PH = groups // 2
    CH = PH * G

    @functools.partial(
        pl.kernel,
        mesh=_mesh,
        compiler_params=_cp,
        out_type=jax.ShapeDtypeStruct((2 * ET_PAD,), jnp.float32),
        scratch_types=[
            pltpu.VMEM((CH,), jnp.int32),
            pltpu.VMEM((CH,), jnp.int32),
            pltpu.VMEM((CH,), jnp.float32),
            pltpu.VMEM((G, 128), jnp.float32),
            pltpu.VMEM((G, 128), jnp.float32),
            pltpu.VMEM((G, 128), jnp.float32),
            pltpu.VMEM((G, 128), jnp.float32),
            pltpu.VMEM((128,), jnp.float32),
            pltpu.VMEM((G * 16,), jnp.float32),
            pltpu.SemaphoreType.DMA,
            pltpu.SemaphoreType.DMA,
            pltpu.SemaphoreType.DMA,
            pltpu.SemaphoreType.DMA,
        ],
    )
    def k(xl_h, xr_h, src_h, dst_h, att_h, plog_h,
          sbuf, dbuf, plogb, u0, v0, u1, v1, attv, accb,
          su0, sv0, su1, sv1):
        c = lax.axis_index("c")
        t = lax.axis_index("s")
        cN = c * N
        pltpu.sync_copy(att_h.at[pl.ds(c * 128, 128)], attv)
        att_ch = [attv[pl.ds(j * 16, 16)] for j in range(8)]

        def issue(g, ub, vb, semu, semv):
            sl = pl.ds(g * G, G)
            pltpu.async_copy(xl_h.at[sbuf.at[sl]], ub, semu)
            pltpu.async_copy(xr_h.at[dbuf.at[sl]], vb, semv)

        def wait(ub, vb, semu, semv):
            pltpu.make_async_copy(xl_h.at[pl.ds(0, G)], ub, semu).wait()
            pltpu.make_async_copy(xr_h.at[pl.ds(0, G)], vb, semv).wait()

        def compute(g, ub, vb):
            @pl.loop(0, G)
            def _(e):
                sl0 = pl.ds(0, 16)
                tv = ub[e, sl0] + vb[e, sl0]
                acc = jnp.maximum(tv, 0.2 * tv) * att_ch[0]
                for hc in range(1, 8):
                    sl = pl.ds(hc * 16, 16)
                    tv = ub[e, sl] + vb[e, sl]
                    acc = acc + jnp.maximum(tv, 0.2 * tv) * att_ch[hc]
                accb[pl.ds(e * 16, 16)] = acc

            @pl.loop(0, G // 16)
            def _(eg):
                lanes = lax.iota(jnp.int32, 16) * 16 + eg * 256
                lg = plsc.load_gather(accb, [lanes])
                for j in range(1, 16):
                    lg = lg + plsc.load_gather(accb, [lanes + j])
                plogb[pl.ds(g * G + eg * 16, 16)] = lg

        for p in range(2):
            base0 = t * (groups * G) + p * CH
            pltpu.sync_copy(src_h.at[pl.ds(base0, CH)], sbuf)
            pltpu.sync_copy(dst_h.at[pl.ds(base0, CH)], dbuf)

            @pl.loop(0, CH // 16)
            def _(i):
                sl = pl.ds(i * 16, 16)
                sbuf[sl] = sbuf[sl] + cN
                dbuf[sl] = dbuf[sl] + cN

            issue(0, u0, v0, su0, sv0)

            @pl.loop(0, PH // 2)
            def _(it):
                g0 = it * 2
                issue(g0 + 1, u1, v1, su1, sv1)
                wait(u0, v0, su0, sv0)
                compute(g0, u0, v0)
                issue(g0 + 2, u0, v0, su0, sv0)
                wait(u1, v1, su1, sv1)
                compute(g0 + 1, u1, v1)

            wait(u0, v0, su0, sv0)
            compute(PH - 1, u0, v0)
            pltpu.sync_copy(plogb, plog_h.at[pl.ds(c * ET_PAD + base0, CH)])

    return k(xl2, xr2, src, dst, att)


def _combine_max_body(p_ref, lo_ref, m_ref):
    s = p_ref[0] + p_ref[1]
    r = lax.broadcasted_iota(jnp.int32, s.shape, 0)
    cidx = lax.broadcasted_iota(jnp.int32, s.shape, 1)
    gidx = r * 128 + cidx
    s = jnp.where(gidx < ET, s, NEG)
    lo_ref[...] = s
    m_ref[...] = jnp.broadcast_to(jnp.max(s), (8, 128))


def _combine_max(plog3):
    return pl.pallas_call(
        _combine_max_body,
        out_shape=[
            jax.ShapeDtypeStruct((ET_PAD // 128, 128), jnp.float32),
            jax.ShapeDtypeStruct((8, 128), jnp.float32),
        ],
    )(plog3)


def _sc_aggregate(xl2, logits, src, dst, m16):
    groups = ET_PAD // (NTILE * G)
    rows_per_tile = NPAD // NTILE

    @functools.partial(
        pl.kernel,
        mesh=_mesh,
        compiler_params=_cp,
        out_type=(
            jax.ShapeDtypeStruct((2 * NPAD, 128), jnp.float32),
            jax.ShapeDtypeStruct((NPAD,), jnp.float32),
        ),
        scratch_types=[
            pltpu.VMEM_SHARED((NPAD, 128), jnp.float32),
            pltpu.VMEM_SHARED((NPAD,), jnp.float32),
            pltpu.VMEM((G,), jnp.int32),
            pltpu.VMEM((G,), jnp.int32),
            pltpu.VMEM((G,), jnp.float32),
            pltpu.VMEM((G,), jnp.float32),
            pltpu.VMEM((G, 128), jnp.float32),
            pltpu.VMEM((128, 128), jnp.float32),
            pltpu.VMEM((rows_per_tile,), jnp.float32),
            pltpu.VMEM((16,), jnp.float32),
            pltpu.SemaphoreType.DMA,
        ],
    )
    def k(xl_h, lg_h, src_h, dst_h, m_h, out_h, den_h,
          oacc, dacc, sadj, didx, lgv, exv, rows, zb, zd, mv, sem):
        c = lax.axis_index("c")
        t = lax.axis_index("s")
        cN = c * N
        pltpu.sync_copy(m_h, mv)
        mvec = mv[...]

        @pl.loop(0, 128)
        def _(r):
            for j in range(8):
                zb[r, pl.ds(j * 16, 16)] = jnp.zeros((16,), jnp.float32)

        @pl.loop(0, rows_per_tile // 16)
        def _(r):
            zd[pl.ds(r * 16, 16)] = jnp.zeros((16,), jnp.float32)

        r0 = t * rows_per_tile
        for kk in range(rows_per_tile // 128):
            pltpu.sync_copy(zb, oacc.at[pl.ds(r0 + kk * 128, 128)])
        pltpu.sync_copy(zd, dacc.at[pl.ds(r0, rows_per_tile)])
        plsc.subcore_barrier()

        tile_base = t * (groups * G)

        @pl.loop(0, groups)
        def _(g):
            base = tile_base + g * G
            pltpu.sync_copy(src_h.at[pl.ds(base, G)], sadj)
            pltpu.sync_copy(dst_h.at[pl.ds(base, G)], didx)
            pltpu.sync_copy(lg_h.at[pl.ds(base, G)], lgv)
            for j in range(G // 16):
                sl = pl.ds(j * 16, 16)
                sadj[sl] = sadj[sl] + cN
                exv[sl] = jnp.exp(lgv[sl] - mvec)
            pltpu.async_copy(xl_h.at[sadj], rows, sem).wait()

            @pl.loop(0, G)
            def _(e):
                ee = plsc.load_gather(exv, [jnp.full((16,), e, jnp.int32)])
                for hc in range(8):
                    sl = pl.ds(hc * 16, 16)
                    rows[e, sl] = rows[e, sl] * ee

            pltpu.sync_copy(rows, oacc.at[didx], add=True)
            pltpu.sync_copy(exv, dacc.at[didx], add=True)

        plsc.subcore_barrier()
        pltpu.sync_copy(oacc.at[pl.ds(r0, rows_per_tile)],
                        out_h.at[pl.ds(c * NPAD + r0, rows_per_tile)])

        @pl.when(c == 0)
        def _():
            pltpu.sync_copy(dacc.at[pl.ds(r0, rows_per_tile)],
                            den_h.at[pl.ds(r0, rows_per_tile)])

    return k(xl2, logits, src, dst, m16)


def _bn_stats_body(o_ref, d_ref, b_ref, e_ref, s_ref, ss_ref):
    i = pl.program_id(1)
    o = o_ref[0] / (d_ref[0, 0][:, None] + 1e-16) + b_ref[0, 0][None, :]
    r = lax.broadcasted_iota(jnp.int32, o.shape, 0) + i * o.shape[0]
    o = jnp.where(r < N, o, 0.0)
    e_ref[0] = o

    @pl.when(i == 0)
    def _():
        s_ref[0, 0] = jnp.zeros((128,), jnp.float32)
        ss_ref[0, 0] = jnp.zeros((128,), jnp.float32)
    s_ref[0, 0] += jnp.sum(o, axis=0)
    ss_ref[0, 0] += jnp.sum(o * o, axis=0)


def _bn_stats(out3, denom2, bias2):
    Bn = 1024
    return pl.pallas_call(
        _bn_stats_body,
        grid=(2, NPAD // Bn),
        in_specs=[
            pl.BlockSpec((1, Bn, 128), lambda h, i: (h, i, 0)),
            pl.BlockSpec((1, 1, Bn), lambda h, i: (i, 0, 0)),
            pl.BlockSpec((1, 1, 128), lambda h, i: (h, 0, 0)),
        ],
        out_specs=[
            pl.BlockSpec((1, Bn, 128), lambda h, i: (h, i, 0)),
            pl.BlockSpec((1, 1, 128), lambda h, i: (h, 0, 0)),
            pl.BlockSpec((1, 1, 128), lambda h, i: (h, 0, 0)),
        ],
        out_shape=[
            jax.ShapeDtypeStruct((2, NPAD, 128), jnp.float32),
            jax.ShapeDtypeStruct((2, 1, 128), jnp.float32),
            jax.ShapeDtypeStruct((2, 1, 128), jnp.float32),
        ],
    )(out3, denom2, bias2.reshape(2, 1, 128))


def _bn_apply_body(e_ref, s_ref, ss_ref, g_ref, b_ref, o_ref):
    mean = s_ref[0, 0] / N
    var = ss_ref[0, 0] / N - mean * mean
    scale = g_ref[0, 0][None, :] * lax.rsqrt(var + 1e-5)[None, :]
    o = (e_ref[0] - mean[None, :]) * scale + b_ref[0, 0][None, :]
    o_ref[0] = jnp.maximum(o, 0.0)


def _bn_apply(emb3, s, ss, gamma2, beta2):
    Bn = 1024
    return pl.pallas_call(
        _bn_apply_body,
        grid=(2, NPAD // Bn),
        in_specs=[
            pl.BlockSpec((1, Bn, 128), lambda h, i: (h, i, 0)),
            pl.BlockSpec((1, 1, 128), lambda h, i: (h, 0, 0)),
            pl.BlockSpec((1, 1, 128), lambda h, i: (h, 0, 0)),
            pl.BlockSpec((1, 1, 128), lambda h, i: (h, 0, 0)),
            pl.BlockSpec((1, 1, 128), lambda h, i: (h, 0, 0)),
        ],
        out_specs=pl.BlockSpec((1, Bn, 128), lambda h, i: (h, i, 0)),
        out_shape=jax.ShapeDtypeStruct((2, NPAD, 128), jnp.float32),
    )(emb3, s, ss, gamma2.reshape(2, 1, 128), beta2.reshape(2, 1, 128))


def _sc_scores(embs2, sl_idx, tl_idx):
    groups = EL_PAD // (NTILE * G)

    @functools.partial(
        pl.kernel,
        mesh=_mesh,
        compiler_params=_cp,
        out_type=jax.ShapeDtypeStruct((2 * EL_PAD,), jnp.float32),
        scratch_types=[
            pltpu.VMEM((G,), jnp.int32),
            pltpu.VMEM((G,), jnp.int32),
            pltpu.VMEM((G, 128), jnp.float32),
            pltpu.VMEM((G, 128), jnp.float32),
            pltpu.VMEM((G * 16,), jnp.float32),
            pltpu.VMEM((G,), jnp.float32),
            pltpu.SemaphoreType.DMA,
            pltpu.SemaphoreType.DMA,
        ],
    )
    def k(emb_h, s_h, t_h, ps_h,
          sadj, tadj, srows, trows, accb, psv, sem1, sem2):
        c = lax.axis_index("c")
        t = lax.axis_index("s")
        cN = c * NPAD
        tile_base = t * (groups * G)

        @pl.loop(0, groups)
        def _(g):
            base = tile_base + g * G
            pltpu.sync_copy(s_h.at[pl.ds(base, G)], sadj)
            pltpu.sync_copy(t_h.at[pl.ds(base, G)], tadj)
            for j in range(G // 16):
                sl = pl.ds(j * 16, 16)
                sadj[sl] = sadj[sl] + cN
                tadj[sl] = tadj[sl] + cN
            cp1 = pltpu.async_copy(emb_h.at[sadj], srows, sem1)
            cp2 = pltpu.async_copy(emb_h.at[tadj], trows, sem2)
            cp1.wait()
            cp2.wait()

            @pl.loop(0, G)
            def _(e):
                acc = jnp.zeros((16,), jnp.float32)
                for hc in range(8):
                    sl = pl.ds(hc * 16, 16)
                    acc = acc + srows[e, sl] * trows[e, sl]
                accb[pl.ds(e * 16, 16)] = acc

            @pl.loop(0, G // 16)
            def _(eg):
                lanes = lax.iota(jnp.int32, 16) * 16 + eg * 256
                ps = jnp.zeros((16,), jnp.float32)
                for j in range(16):
                    ps = ps + plsc.load_gather(accb, [lanes + j])
                psv[pl.ds(eg * 16, 16)] = ps

            pltpu.sync_copy(psv, ps_h.at[pl.ds(c * EL_PAD + base, G)])

    return k(embs2, sl_idx, tl_idx)


def _combine_scores_body(p_ref, o_ref):
    o_ref[...] = p_ref[0] + p_ref[1]


def _combine_scores(ps3):
    return pl.pallas_call(
        _combine_scores_body,
        out_shape=jax.ShapeDtypeStruct((EL_PAD // 128, 128), jnp.float32),
    )(ps3)


@jax.jit
def _run(x, edge_index, edge_label_index, W_l, b_l, W_r, b_r, att, bias,
         gamma, beta):
    loop = jnp.arange(N, dtype=jnp.int32)
    pad_e = jnp.zeros((ET_PAD - ET,), jnp.int32)
    src_f = jnp.concatenate([edge_index[0], loop, pad_e])
    dst_f = jnp.concatenate([edge_index[1], loop, pad_e])

    xl3, xr3 = _dense_pre(x, W_l, b_l, W_r, b_r)
    xl2 = xl3.reshape(2 * N, 128)
    xr2 = xr3.reshape(2 * N, 128)

    plog = _sc_logits(xl2, xr2, src_f, dst_f, att)
    logits3, m = _combine_max(plog.reshape(2, ET_PAD // 128, 128))
    m16 = jnp.full((16,), m[0, 0], jnp.float32)

    out2, denom = _sc_aggregate(xl2, logits3.reshape(ET_PAD), src_f, dst_f, m16)
    emb3, s, ss = _bn_stats(out2.reshape(2, NPAD, 128),
                            denom.reshape(NPAD // 1024, 1, 1024),
                            bias.reshape(2, 128))
    embs3 = _bn_apply(emb3, s, ss, gamma.reshape(2, 128), beta.reshape(2, 128))

    pad_l = jnp.zeros((EL_PAD - EL,), jnp.int32)
    sl_idx = jnp.concatenate([edge_label_index[0], pad_l])
    tl_idx = jnp.concatenate([edge_label_index[1], pad_l])
    ps = _sc_scores(embs3.reshape(2 * NPAD, 128), sl_idx, tl_idx)
    sc3 = _combine_scores(ps.reshape(2, EL_PAD // 128, 128))
    return sc3.reshape(EL_PAD)[:EL]


def kernel(x, edge_index, edge_label_index, W_l, b_l, W_r, b_r, att, bias,
           gamma, beta):
    return _run(x, edge_index, edge_label_index, W_l, b_l, W_r, b_r, att,
                bias, gamma, beta)


# speedup vs baseline: 5.0194x; 1.1102x over previous
import dataclasses
import functools

import jax
import jax.numpy as jnp
from jax import lax
from jax.experimental import pallas as pl
from jax.experimental.pallas import tpu as pltpu
from jax.experimental.pallas import tpu_sc as plsc

N = 10000
D = 128
H = 256
E = 320000
EL = 320000
ET = E + N
G = 128
NTILE = 16
ET_PAD = 16 * G * 162
EL_PAD = 16 * G * 157
NPAD = 10240
NEG = -3.0e38

_mesh = plsc.VectorSubcoreMesh(core_axis_name="c", subcore_axis_name="s")

_cp = pltpu.CompilerParams()
if "needs_layout_passes" in pltpu.CompilerParams.__dataclass_fields__:
    _cp = dataclasses.replace(_cp, needs_layout_passes=False)


def _dense_body(x_ref, wl_ref, bl_ref, wr_ref, br_ref, ol_ref, or_ref):
    xb = x_ref[...]
    a = jnp.dot(xb, wl_ref[...], preferred_element_type=jnp.float32) + bl_ref[...]
    ol_ref[0] = a[:, :128]
    ol_ref[1] = a[:, 128:]
    b = jnp.dot(xb, wr_ref[...], preferred_element_type=jnp.float32) + br_ref[...]
    or_ref[0] = b[:, :128]
    or_ref[1] = b[:, 128:]


def _dense_pre(x, W_l, b_l, W_r, b_r):
    Bn = 1000
    return pl.pallas_call(
        _dense_body,
        grid=(N // Bn,),
        in_specs=[
            pl.BlockSpec((Bn, D), lambda i: (i, 0)),
            pl.BlockSpec((D, H), lambda i: (0, 0)),
            pl.BlockSpec((1, H), lambda i: (0, 0)),
            pl.BlockSpec((D, H), lambda i: (0, 0)),
            pl.BlockSpec((1, H), lambda i: (0, 0)),
        ],
        out_specs=[
            pl.BlockSpec((2, Bn, 128), lambda i: (0, i, 0)),
            pl.BlockSpec((2, Bn, 128), lambda i: (0, i, 0)),
        ],
        out_shape=[
            jax.ShapeDtypeStruct((2, N, 128), jnp.float32),
            jax.ShapeDtypeStruct((2, N, 128), jnp.float32),
        ],
    )(x, W_l, b_l.reshape(1, H), W_r, b_r.reshape(1, H))


def _sc_logits(xl2, xr2, src, dst, att):
    groups = ET_PAD // (NTILE * G)
    PH = groups // 2
    CH = PH * G

    @functools.partial(
        pl.kernel,
        mesh=_mesh,
        compiler_params=_cp,
        out_type=jax.ShapeDtypeStruct((2 * ET_PAD,), jnp.float32),
        scratch_types=[
            pltpu.VMEM((CH,), jnp.int32),
            pltpu.VMEM((CH,), jnp.int32),
            pltpu.VMEM((CH,), jnp.float32),
            pltpu.VMEM((G, 128), jnp.float32),
            pltpu.VMEM((G, 128), jnp.float32),
            pltpu.VMEM((G, 128), jnp.float32),
            pltpu.VMEM((G, 128), jnp.float32),
            pltpu.VMEM((128,), jnp.float32),
            pltpu.VMEM((G * 16,), jnp.float32),
            pltpu.SemaphoreType.DMA,
            pltpu.SemaphoreType.DMA,
            pltpu.SemaphoreType.DMA,
            pltpu.SemaphoreType.DMA,
        ],
    )
    def k(xl_h, xr_h, src_h, dst_h, att_h, plog_h,
          sbuf, dbuf, plogb, u0, v0, u1, v1, attv, accb,
          su0, sv0, su1, sv1):
        c = lax.axis_index("c")
        t = lax.axis_index("s")
        cN = c * N
        pltpu.sync_copy(att_h.at[pl.ds(c * 128, 128)], attv)
        att_ch = [attv[pl.ds(j * 16, 16)] for j in range(8)]

        def issue(g, ub, vb, semu, semv):
            sl = pl.ds(g * G, G)
            pltpu.async_copy(xl_h.at[sbuf.at[sl]], ub, semu)
            pltpu.async_copy(xr_h.at[dbuf.at[sl]], vb, semv)

        def wait(ub, vb, semu, semv):
            pltpu.make_async_copy(xl_h.at[pl.ds(0, G)], ub, semu).wait()
            pltpu.make_async_copy(xr_h.at[pl.ds(0, G)], vb, semv).wait()

        def compute(g, ub, vb):
            @pl.loop(0, G)
            def _(e):
                sl0 = pl.ds(0, 16)
                tv = ub[e, sl0] + vb[e, sl0]
                acc = jnp.maximum(tv, 0.2 * tv) * att_ch[0]
                for hc in range(1, 8):
                    sl = pl.ds(hc * 16, 16)
                    tv = ub[e, sl] + vb[e, sl]
                    acc = acc + jnp.maximum(tv, 0.2 * tv) * att_ch[hc]
                accb[pl.ds(e * 16, 16)] = acc

            @pl.loop(0, G // 16)
            def _(eg):
                lanes = lax.iota(jnp.int32, 16) * 16 + eg * 256
                lg = plsc.load_gather(accb, [lanes])
                for j in range(1, 16):
                    lg = lg + plsc.load_gather(accb, [lanes + j])
                plogb[pl.ds(g * G + eg * 16, 16)] = lg

        for p in range(2):
            base0 = t * (groups * G) + p * CH
            pltpu.sync_copy(src_h.at[pl.ds(base0, CH)], sbuf)
            pltpu.sync_copy(dst_h.at[pl.ds(base0, CH)], dbuf)

            @pl.loop(0, CH // 16)
            def _(i):
                sl = pl.ds(i * 16, 16)
                sbuf[sl] = sbuf[sl] + cN
                dbuf[sl] = dbuf[sl] + cN

            issue(0, u0, v0, su0, sv0)

            @pl.loop(0, PH // 2)
            def _(it):
                g0 = it * 2
                issue(g0 + 1, u1, v1, su1, sv1)
                wait(u0, v0, su0, sv0)
                compute(g0, u0, v0)
                issue(g0 + 2, u0, v0, su0, sv0)
                wait(u1, v1, su1, sv1)
                compute(g0 + 1, u1, v1)

            wait(u0, v0, su0, sv0)
            compute(PH - 1, u0, v0)
            pltpu.sync_copy(plogb, plog_h.at[pl.ds(c * ET_PAD + base0, CH)])

    return k(xl2, xr2, src, dst, att)


def _combine_max_body(p_ref, lo_ref, m_ref):
    s = p_ref[0] + p_ref[1]
    r = lax.broadcasted_iota(jnp.int32, s.shape, 0)
    cidx = lax.broadcasted_iota(jnp.int32, s.shape, 1)
    gidx = r * 128 + cidx
    s = jnp.where(gidx < ET, s, NEG)
    lo_ref[...] = s
    m_ref[...] = jnp.broadcast_to(jnp.max(s), (8, 128))


def _combine_max(plog3):
    return pl.pallas_call(
        _combine_max_body,
        out_shape=[
            jax.ShapeDtypeStruct((ET_PAD // 128, 128), jnp.float32),
            jax.ShapeDtypeStruct((8, 128), jnp.float32),
        ],
    )(plog3)


def _sc_aggregate(xl2, logits, src, dst, m16):
    groups = ET_PAD // (NTILE * G)
    rows_per_tile = NPAD // NTILE

    @functools.partial(
        pl.kernel,
        mesh=_mesh,
        compiler_params=_cp,
        out_type=(
            jax.ShapeDtypeStruct((2 * NPAD, 128), jnp.float32),
            jax.ShapeDtypeStruct((NPAD,), jnp.float32),
        ),
        scratch_types=[
            pltpu.VMEM_SHARED((NPAD, 128), jnp.float32),
            pltpu.VMEM_SHARED((NPAD,), jnp.float32),
            pltpu.VMEM((G,), jnp.int32),
            pltpu.VMEM((G,), jnp.int32),
            pltpu.VMEM((G,), jnp.float32),
            pltpu.VMEM((G,), jnp.float32),
            pltpu.VMEM((G, 128), jnp.float32),
            pltpu.VMEM((128, 128), jnp.float32),
            pltpu.VMEM((rows_per_tile,), jnp.float32),
            pltpu.VMEM((16,), jnp.float32),
            pltpu.SemaphoreType.DMA,
        ],
    )
    def k(xl_h, lg_h, src_h, dst_h, m_h, out_h, den_h,
          oacc, dacc, sadj, didx, lgv, exv, rows, zb, zd, mv, sem):
        c = lax.axis_index("c")
        t = lax.axis_index("s")
        cN = c * N
        pltpu.sync_copy(m_h, mv)
        mvec = mv[...]

        @pl.loop(0, 128)
        def _(r):
            for j in range(8):
                zb[r, pl.ds(j * 16, 16)] = jnp.zeros((16,), jnp.float32)

        @pl.loop(0, rows_per_tile // 16)
        def _(r):
            zd[pl.ds(r * 16, 16)] = jnp.zeros((16,), jnp.float32)

        r0 = t * rows_per_tile
        for kk in range(rows_per_tile // 128):
            pltpu.sync_copy(zb, oacc.at[pl.ds(r0 + kk * 128, 128)])
        pltpu.sync_copy(zd, dacc.at[pl.ds(r0, rows_per_tile)])
        plsc.subcore_barrier()

        tile_base = t * (groups * G)

        @pl.loop(0, groups)
        def _(g):
            base = tile_base + g * G
            pltpu.sync_copy(src_h.at[pl.ds(base, G)], sadj)
            pltpu.sync_copy(dst_h.at[pl.ds(base, G)], didx)
            pltpu.sync_copy(lg_h.at[pl.ds(base, G)], lgv)
            for j in range(G // 16):
                sl = pl.ds(j * 16, 16)
                sadj[sl] = sadj[sl] + cN
                exv[sl] = jnp.exp(lgv[sl] - mvec)
            pltpu.async_copy(xl_h.at[sadj], rows, sem).wait()

            @pl.loop(0, G)
            def _(e):
                ee = plsc.load_gather(exv, [jnp.full((16,), e, jnp.int32)])
                for hc in range(8):
                    sl = pl.ds(hc * 16, 16)
                    rows[e, sl] = rows[e, sl] * ee

            pltpu.sync_copy(rows, oacc.at[didx], add=True)
            pltpu.sync_copy(exv, dacc.at[didx], add=True)

        plsc.subcore_barrier()
        pltpu.sync_copy(oacc.at[pl.ds(r0, rows_per_tile)],
                        out_h.at[pl.ds(c * NPAD + r0, rows_per_tile)])

        @pl.when(c == 0)
        def _():
            pltpu.sync_copy(dacc.at[pl.ds(r0, rows_per_tile)],
                            den_h.at[pl.ds(r0, rows_per_tile)])

    return k(xl2, logits, src, dst, m16)


def _bn_stats_body(o_ref, d_ref, b_ref, e_ref, s_ref, ss_ref):
    i = pl.program_id(1)
    o = o_ref[0] / (d_ref[0, 0][:, None] + 1e-16) + b_ref[0, 0][None, :]
    r = lax.broadcasted_iota(jnp.int32, o.shape, 0) + i * o.shape[0]
    o = jnp.where(r < N, o, 0.0)
    e_ref[0] = o

    @pl.when(i == 0)
    def _():
        s_ref[0, 0] = jnp.zeros((128,), jnp.float32)
        ss_ref[0, 0] = jnp.zeros((128,), jnp.float32)
    s_ref[0, 0] += jnp.sum(o, axis=0)
    ss_ref[0, 0] += jnp.sum(o * o, axis=0)


def _bn_stats(out3, denom2, bias2):
    Bn = 1024
    return pl.pallas_call(
        _bn_stats_body,
        grid=(2, NPAD // Bn),
        in_specs=[
            pl.BlockSpec((1, Bn, 128), lambda h, i: (h, i, 0)),
            pl.BlockSpec((1, 1, Bn), lambda h, i: (i, 0, 0)),
            pl.BlockSpec((1, 1, 128), lambda h, i: (h, 0, 0)),
        ],
        out_specs=[
            pl.BlockSpec((1, Bn, 128), lambda h, i: (h, i, 0)),
            pl.BlockSpec((1, 1, 128), lambda h, i: (h, 0, 0)),
            pl.BlockSpec((1, 1, 128), lambda h, i: (h, 0, 0)),
        ],
        out_shape=[
            jax.ShapeDtypeStruct((2, NPAD, 128), jnp.float32),
            jax.ShapeDtypeStruct((2, 1, 128), jnp.float32),
            jax.ShapeDtypeStruct((2, 1, 128), jnp.float32),
        ],
    )(out3, denom2, bias2.reshape(2, 1, 128))


def _bn_apply_body(e_ref, s_ref, ss_ref, g_ref, b_ref, o_ref):
    mean = s_ref[0, 0] / N
    var = ss_ref[0, 0] / N - mean * mean
    scale = g_ref[0, 0][None, :] * lax.rsqrt(var + 1e-5)[None, :]
    o = (e_ref[0] - mean[None, :]) * scale + b_ref[0, 0][None, :]
    o_ref[0] = jnp.maximum(o, 0.0)


def _bn_apply(emb3, s, ss, gamma2, beta2):
    Bn = 1024
    return pl.pallas_call(
        _bn_apply_body,
        grid=(2, NPAD // Bn),
        in_specs=[
            pl.BlockSpec((1, Bn, 128), lambda h, i: (h, i, 0)),
            pl.BlockSpec((1, 1, 128), lambda h, i: (h, 0, 0)),
            pl.BlockSpec((1, 1, 128), lambda h, i: (h, 0, 0)),
            pl.BlockSpec((1, 1, 128), lambda h, i: (h, 0, 0)),
            pl.BlockSpec((1, 1, 128), lambda h, i: (h, 0, 0)),
        ],
        out_specs=pl.BlockSpec((1, Bn, 128), lambda h, i: (h, i, 0)),
        out_shape=jax.ShapeDtypeStruct((2, NPAD, 128), jnp.float32),
    )(emb3, s, ss, gamma2.reshape(2, 1, 128), beta2.reshape(2, 1, 128))


def _sc_scores(embs2, sl_idx, tl_idx):
    groups = EL_PAD // (NTILE * G)
    rows_per_tile = NPAD // NTILE

    @functools.partial(
        pl.kernel,
        mesh=_mesh,
        compiler_params=_cp,
        out_type=jax.ShapeDtypeStruct((2 * EL_PAD,), jnp.float32),
        scratch_types=[
            pltpu.VMEM_SHARED((NPAD, 128), jnp.float32),
            pltpu.VMEM((G,), jnp.int32),
            pltpu.VMEM((G,), jnp.int32),
            pltpu.VMEM((G, 128), jnp.float32),
            pltpu.VMEM((G, 128), jnp.float32),
            pltpu.VMEM((G * 16,), jnp.float32),
            pltpu.VMEM((G,), jnp.float32),
            pltpu.SemaphoreType.DMA,
            pltpu.SemaphoreType.DMA,
        ],
    )
    def k(emb_h, s_h, t_h, ps_h,
          etab, sadj, tadj, srows, trows, accb, psv, sem1, sem2):
        c = lax.axis_index("c")
        t = lax.axis_index("s")
        tile_base = t * (groups * G)

        r0 = t * rows_per_tile
        pltpu.sync_copy(emb_h.at[pl.ds(c * NPAD + r0, rows_per_tile)],
                        etab.at[pl.ds(r0, rows_per_tile)])
        plsc.subcore_barrier()

        @pl.loop(0, groups)
        def _(g):
            base = tile_base + g * G
            pltpu.sync_copy(s_h.at[pl.ds(base, G)], sadj)
            pltpu.sync_copy(t_h.at[pl.ds(base, G)], tadj)
            cp1 = pltpu.async_copy(etab.at[sadj], srows, sem1)
            cp2 = pltpu.async_copy(etab.at[tadj], trows, sem2)
            cp1.wait()
            cp2.wait()

            @pl.loop(0, G)
            def _(e):
                acc = jnp.zeros((16,), jnp.float32)
                for hc in range(8):
                    sl = pl.ds(hc * 16, 16)
                    acc = acc + srows[e, sl] * trows[e, sl]
                accb[pl.ds(e * 16, 16)] = acc

            @pl.loop(0, G // 16)
            def _(eg):
                lanes = lax.iota(jnp.int32, 16) * 16 + eg * 256
                ps = jnp.zeros((16,), jnp.float32)
                for j in range(16):
                    ps = ps + plsc.load_gather(accb, [lanes + j])
                psv[pl.ds(eg * 16, 16)] = ps

            pltpu.sync_copy(psv, ps_h.at[pl.ds(c * EL_PAD + base, G)])

    return k(embs2, sl_idx, tl_idx)


def _combine_scores_body(p_ref, o_ref):
    o_ref[...] = p_ref[0] + p_ref[1]


def _combine_scores(ps3):
    return pl.pallas_call(
        _combine_scores_body,
        out_shape=jax.ShapeDtypeStruct((EL_PAD // 128, 128), jnp.float32),
    )(ps3)


@jax.jit
def _run(x, edge_index, edge_label_index, W_l, b_l, W_r, b_r, att, bias,
         gamma, beta):
    loop = jnp.arange(N, dtype=jnp.int32)
    pad_e = jnp.zeros((ET_PAD - ET,), jnp.int32)
    src_f = jnp.concatenate([edge_index[0], loop, pad_e])
    dst_f = jnp.concatenate([edge_index[1], loop, pad_e])

    xl3, xr3 = _dense_pre(x, W_l, b_l, W_r, b_r)
    xl2 = xl3.reshape(2 * N, 128)
    xr2 = xr3.reshape(2 * N, 128)

    plog = _sc_logits(xl2, xr2, src_f, dst_f, att)
    logits3, m = _combine_max(plog.reshape(2, ET_PAD // 128, 128))
    m16 = jnp.full((16,), m[0, 0], jnp.float32)

    out2, denom = _sc_aggregate(xl2, logits3.reshape(ET_PAD), src_f, dst_f, m16)
    emb3, s, ss = _bn_stats(out2.reshape(2, NPAD, 128),
                            denom.reshape(NPAD // 1024, 1, 1024),
                            bias.reshape(2, 128))
    embs3 = _bn_apply(emb3, s, ss, gamma.reshape(2, 128), beta.reshape(2, 128))

    pad_l = jnp.zeros((EL_PAD - EL,), jnp.int32)
    sl_idx = jnp.concatenate([edge_label_index[0], pad_l])
    tl_idx = jnp.concatenate([edge_label_index[1], pad_l])
    ps = _sc_scores(embs3.reshape(2 * NPAD, 128), sl_idx, tl_idx)
    sc3 = _combine_scores(ps.reshape(2, EL_PAD // 128, 128))
    return sc3.reshape(EL_PAD)[:EL]


def kernel(x, edge_index, edge_label_index, W_l, b_l, W_r, b_r, att, bias,
           gamma, beta):
    return _run(x, edge_index, edge_label_index, W_l, b_l, W_r, b_r, att,
                bias, gamma, beta)
